# baseline (speedup 1.0000x reference)
import functools

import jax
import jax.numpy as jnp
import numpy as np
from jax import lax
from jax.experimental import pallas as pl
from jax.experimental.pallas import tpu as pltpu
from jax.experimental.pallas import tpu_sc as plsc

_FACTOR = 3
_B, _L, _H, _D = 1, 2048, 12, 64
_SAMPLE_K = 2 * _FACTOR * int(np.ceil(np.log(_L)))
_NTOP = _FACTOR * int(np.ceil(np.log(_L)))
_SCALE = 1.0 / float(np.sqrt(_D))

_BQ = 512
_NQC = _L // _BQ

_NWORK = 32
_TOTAL = _H * _L * _L
_PER_W = _TOTAL // _NWORK
_ZCHUNK = 65536


def _threefry_raw(k1, k2, x1, x2):
    u32 = np.uint32
    def rotl(x, d):
        return (x << u32(d)) | (x >> u32(32 - d))
    ks = [u32(k1), u32(k2), u32(k1) ^ u32(k2) ^ u32(0x1BD11BDA)]
    rotations = [(13, 15, 26, 6), (17, 29, 16, 24)]
    x = [x1.astype(np.uint32) + ks[0], x2.astype(np.uint32) + ks[1]]
    for i in range(5):
        for r in rotations[i % 2]:
            x[0] = x[0] + x[1]
            x[1] = rotl(x[1], r)
            x[1] = x[0] ^ x[1]
        x[0] = x[0] + ks[(i + 1) % 3]
        x[1] = x[1] + ks[(i + 2) % 3] + u32(i + 1)
    return x[0], x[1]


def _build_count():
    k1, k2 = np.uint32(0), np.uint32(42)
    b1, b2 = _threefry_raw(k1, k2, np.zeros(2, np.uint32),
                           np.arange(2, dtype=np.uint32))
    n = _L * _SAMPLE_K
    o1, o2 = _threefry_raw(b1[1], b2[1], np.zeros(n, np.uint32),
                           np.arange(n, dtype=np.uint32))
    idx = ((o1 ^ o2) % np.uint32(_L)).astype(np.int64).reshape(_L, _SAMPLE_K)
    count = np.zeros((_L, _L), dtype=np.int8)
    np.add.at(count, (np.arange(_L)[:, None], idx), 1)
    return count


_TABLE_CACHE: list = []


def _get_count():
    if not _TABLE_CACHE:
        _TABLE_CACHE.append(_build_count())
    return _TABLE_CACHE[0]


def _compute_body(q_ref, k_ref, c_ref, idx_ref, attn_ref):
    k = k_ref[0]

    m_parts = []
    for qc in range(_NQC):
        q = q_ref[0, qc * _BQ:(qc + 1) * _BQ, :]
        cnt = c_ref[qc * _BQ:(qc + 1) * _BQ, :].astype(jnp.float32)
        s = lax.dot_general(q, k, (((1,), (1,)), ((), ())),
                            precision=lax.Precision.HIGHEST,
                            preferred_element_type=jnp.float32)
        mx = jnp.max(jnp.where(cnt > 0, s, -jnp.inf), axis=1)
        sm = jnp.sum(s * cnt, axis=1) / _L
        m_parts.append((mx - sm).reshape(1, _BQ))
    mcur = jnp.concatenate(m_parts, axis=1)

    iota = lax.broadcasted_iota(jnp.int32, (1, _L), 1)
    oh_rows = []
    for u in range(_NTOP):
        mxv = jnp.max(mcur)
        idx_u = jnp.min(jnp.where(mcur == mxv, iota, _L))
        sel = iota == idx_u
        oh_rows.append(sel.astype(jnp.float32))
        mcur = jnp.where(sel, -jnp.inf, mcur)
    oh = jnp.concatenate(oh_rows, axis=0)
    lane = lax.broadcasted_iota(jnp.int32, (_NTOP, _L), 1).astype(jnp.float32)
    idx_ref[0, 0, :] = jnp.sum(oh * lane, axis=1).astype(jnp.int32)

    qs = lax.dot_general(oh, q_ref[0], (((1,), (0,)), ((), ())),
                         precision=lax.Precision.HIGHEST,
                         preferred_element_type=jnp.float32)
    s = lax.dot_general(qs, k, (((1,), (1,)), ((), ())),
                        precision=lax.Precision.HIGHEST,
                        preferred_element_type=jnp.float32)
    s = s * _SCALE
    s = s - jnp.max(s, axis=1, keepdims=True)
    e = jnp.exp(s)
    attn_ref[0] = e / jnp.sum(e, axis=1, keepdims=True)


def _memset_body(o_hbm, zbuf):
    wid = lax.axis_index("s") * 2 + lax.axis_index("c")
    base = wid * _PER_W

    def zero_init(i, carry):
        zbuf[pl.ds(i * 16, 16)] = jnp.zeros((16,), jnp.float32)
        return carry

    lax.fori_loop(0, _ZCHUNK // 16, zero_init, 0)

    def push(j, carry):
        pltpu.sync_copy(zbuf, o_hbm.at[pl.ds(base + j * _ZCHUNK, _ZCHUNK)])
        return carry

    lax.fori_loop(0, _PER_W // _ZCHUNK, push, 0)


def _scatter_body(idx_ref, attn_ref, zin_ref, o_ref, sem):
    del zin_ref
    h = pl.program_id(0)
    copies = []
    for u in range(_NTOP):
        row = idx_ref[h, 0, u]
        cp = pltpu.make_async_copy(
            attn_ref.at[0, pl.ds(u, 1), :],
            o_ref.at[h, pl.ds(row, 1), :],
            sem,
        )
        cp.start()
        copies.append(cp)
    for cp in copies:
        cp.wait()


@jax.jit
def kernel(queries, keys):
    q = jnp.transpose(queries[0], (1, 0, 2))
    k = jnp.transpose(keys[0], (1, 0, 2))
    cnt = jnp.asarray(_get_count())

    memset = pl.kernel(
        _memset_body,
        out_type=jax.ShapeDtypeStruct((_TOTAL,), jnp.float32),
        mesh=plsc.VectorSubcoreMesh(core_axis_name="c", subcore_axis_name="s"),
        scratch_types=[pltpu.VMEM((_ZCHUNK,), jnp.float32)],
    )
    zeros_flat = memset()
    zeros = zeros_flat.reshape(_H, _L, _L)

    idx, attn = pl.pallas_call(
        _compute_body,
        grid=(_H,),
        in_specs=[
            pl.BlockSpec((1, _L, _D), lambda h: (h, 0, 0)),
            pl.BlockSpec((1, _L, _D), lambda h: (h, 0, 0)),
            pl.BlockSpec((_L, _L), lambda h: (0, 0)),
        ],
        out_specs=[
            pl.BlockSpec((1, 1, _NTOP), lambda h: (h, 0, 0)),
            pl.BlockSpec((1, _NTOP, _L), lambda h: (h, 0, 0)),
        ],
        out_shape=[
            jax.ShapeDtypeStruct((_H, 1, _NTOP), jnp.int32),
            jax.ShapeDtypeStruct((_H, _NTOP, _L), jnp.float32),
        ],
    )(q, k, cnt)

    out = pl.pallas_call(
        _scatter_body,
        grid=(_H,),
        in_specs=[
            pl.BlockSpec(memory_space=pltpu.SMEM),
            pl.BlockSpec((1, _NTOP, _L), lambda h: (h, 0, 0)),
            pl.BlockSpec(memory_space=pl.ANY),
        ],
        out_specs=pl.BlockSpec(memory_space=pl.ANY),
        out_shape=jax.ShapeDtypeStruct((_H, _L, _L), jnp.float32),
        input_output_aliases={2: 0},
        scratch_shapes=[pltpu.SemaphoreType.DMA],
    )(idx, attn, zeros)

    return out.reshape(_B, _H, _L, _L)


# speedup vs baseline: 1.3140x; 1.3140x over previous
import jax
import jax.numpy as jnp
import numpy as np
from jax import lax
from jax.experimental import pallas as pl
from jax.experimental.pallas import tpu as pltpu

_FACTOR = 3
_B, _L, _H, _D = 1, 2048, 12, 64
_SAMPLE_K = 2 * _FACTOR * int(np.ceil(np.log(_L)))
_NTOP = _FACTOR * int(np.ceil(np.log(_L)))
_SCALE = 1.0 / float(np.sqrt(_D))

_BQ = 512
_NQC = _L // _BQ
_BR = 512
_NRB = _L // _BR


def _threefry_raw(k1, k2, x1, x2):
    u32 = np.uint32
    def rotl(x, d):
        return (x << u32(d)) | (x >> u32(32 - d))
    ks = [u32(k1), u32(k2), u32(k1) ^ u32(k2) ^ u32(0x1BD11BDA)]
    rotations = [(13, 15, 26, 6), (17, 29, 16, 24)]
    x = [x1.astype(np.uint32) + ks[0], x2.astype(np.uint32) + ks[1]]
    for i in range(5):
        for r in rotations[i % 2]:
            x[0] = x[0] + x[1]
            x[1] = rotl(x[1], r)
            x[1] = x[0] ^ x[1]
        x[0] = x[0] + ks[(i + 1) % 3]
        x[1] = x[1] + ks[(i + 2) % 3] + u32(i + 1)
    return x[0], x[1]


def _build_count():
    k1, k2 = np.uint32(0), np.uint32(42)
    b1, b2 = _threefry_raw(k1, k2, np.zeros(2, np.uint32),
                           np.arange(2, dtype=np.uint32))
    n = _L * _SAMPLE_K
    o1, o2 = _threefry_raw(b1[1], b2[1], np.zeros(n, np.uint32),
                           np.arange(n, dtype=np.uint32))
    idx = ((o1 ^ o2) % np.uint32(_L)).astype(np.int64).reshape(_L, _SAMPLE_K)
    count = np.zeros((_L, _L), dtype=np.int8)
    np.add.at(count, (np.arange(_L)[:, None], idx), 1)
    return count


_TABLE_CACHE: list = []


def _get_count():
    if not _TABLE_CACHE:
        _TABLE_CACHE.append(_build_count())
    return _TABLE_CACHE[0]


def _fused_body(q_ref, k_ref, c_ref, o_ref, attn_s, idx_s):
    rb = pl.program_id(1)

    @pl.when(rb == 0)
    def _compute():
        k = k_ref[0]

        m_parts = []
        for qc in range(_NQC):
            q = q_ref[0, qc * _BQ:(qc + 1) * _BQ, :]
            cnt = c_ref[qc * _BQ:(qc + 1) * _BQ, :].astype(jnp.float32)
            s = lax.dot_general(q, k, (((1,), (1,)), ((), ())),
                                precision=lax.Precision.HIGHEST,
                                preferred_element_type=jnp.float32)
            mx = jnp.max(jnp.where(cnt > 0, s, -jnp.inf), axis=1)
            sm = jnp.sum(s * cnt, axis=1) / _L
            m_parts.append((mx - sm).reshape(1, _BQ))
        mcur = jnp.concatenate(m_parts, axis=1)

        iota = lax.broadcasted_iota(jnp.int32, (1, _L), 1)
        oh_rows = []
        for u in range(_NTOP):
            mxv = jnp.max(mcur)
            idx_u = jnp.min(jnp.where(mcur == mxv, iota, _L))
            idx_s[u] = idx_u
            sel = iota == idx_u
            oh_rows.append(sel.astype(jnp.float32))
            mcur = jnp.where(sel, -jnp.inf, mcur)
        oh = jnp.concatenate(oh_rows, axis=0)

        qs = lax.dot_general(oh, q_ref[0], (((1,), (0,)), ((), ())),
                             precision=lax.Precision.HIGHEST,
                             preferred_element_type=jnp.float32)
        s = lax.dot_general(qs, k, (((1,), (1,)), ((), ())),
                            precision=lax.Precision.HIGHEST,
                            preferred_element_type=jnp.float32)
        s = s * _SCALE
        s = s - jnp.max(s, axis=1, keepdims=True)
        e = jnp.exp(s)
        attn_s[...] = e / jnp.sum(e, axis=1, keepdims=True)

    base = rb * _BR
    o_ref[0] = jnp.zeros((_BR, _L), jnp.float32)
    for u in range(_NTOP):
        off = idx_s[u] - base

        @pl.when((off >= 0) & (off < _BR))
        def _copy(off=off, u=u):
            o_ref[0, pl.ds(off, 1), :] = attn_s[pl.ds(u, 1), :]


@jax.jit
def kernel(queries, keys):
    q = jnp.transpose(queries[0], (1, 0, 2))
    k = jnp.transpose(keys[0], (1, 0, 2))
    cnt = jnp.asarray(_get_count())

    out = pl.pallas_call(
        _fused_body,
        grid=(_H, _NRB),
        in_specs=[
            pl.BlockSpec((1, _L, _D), lambda h, rb: (h, 0, 0)),
            pl.BlockSpec((1, _L, _D), lambda h, rb: (h, 0, 0)),
            pl.BlockSpec((_L, _L), lambda h, rb: (0, 0)),
        ],
        out_specs=pl.BlockSpec((1, _BR, _L), lambda h, rb: (h, rb, 0)),
        out_shape=jax.ShapeDtypeStruct((_H, _L, _L), jnp.float32),
        scratch_shapes=[
            pltpu.VMEM((_NTOP, _L), jnp.float32),
            pltpu.SMEM((_NTOP,), jnp.int32),
        ],
    )(q, k, cnt)

    return out.reshape(_B, _H, _L, _L)
